# baseline (speedup 1.0000x reference)
import jax
import jax.numpy as jnp
from jax import lax
from jax.experimental import pallas as pl
from jax.experimental.pallas import tpu as pltpu
from jax.experimental.pallas import tpu_sc as plsc

B, N, D = 4, 8192, 512
P = 100
NUM_CLASSES = 91
ROWS_BLK = 512
NSTEP = N // ROWS_BLK
NCH = N // 128
SUB = ROWS_BLK // 128


def _fused_kernel(x_ref, wt_ref, b_ref, mask_ref, perm_ref,
                  s_scr, pos_scr, msk_scr):
    bi = pl.program_id(0)
    i = pl.program_id(1)
    x = x_ref[0]
    logits = jnp.dot(x, wt_ref[...], preferred_element_type=jnp.float32)
    logits = logits + b_ref[0]
    m = jnp.max(logits, axis=-1)
    s = jax.nn.sigmoid(m)
    soft = jax.nn.sigmoid((s - 0.5) / 0.1)
    hard = (s > 0.5).astype(jnp.float32)
    mask_ref[0, 0, :] = hard + soft - soft
    s_scr[pl.ds(i * SUB, SUB), :] = s.reshape(SUB, 128)

    @pl.when(i == NSTEP - 1)
    def _select():
        sall = s_scr[...]
        valid = sall > 0.5
        vf = valid.astype(jnp.float32)

        tri128 = (lax.broadcasted_iota(jnp.int32, (128, 128), 0)
                  <= lax.broadcasted_iota(jnp.int32, (128, 128), 1)
                  ).astype(jnp.float32)
        rowcum = jnp.dot(vf, tri128, preferred_element_type=jnp.float32)
        row_tot = rowcum[:, 127].reshape(1, NCH)
        tri64 = (lax.broadcasted_iota(jnp.int32, (NCH, NCH), 0)
                 <= lax.broadcasted_iota(jnp.int32, (NCH, NCH), 1)
                 ).astype(jnp.float32)
        inc = jnp.dot(row_tot, tri64, preferred_element_type=jnp.float32)
        excl = (inc - row_tot).reshape(NCH, 1)
        pos = rowcum + excl
        num_valid = inc[0, NCH - 1].astype(jnp.int32)

        pos_scr[...] = pos
        msk_scr[...] = jnp.where(valid, 2.0, sall)

        jlane = lax.broadcasted_iota(
            jnp.int32, (1, 128), 1).astype(jnp.float32)

        def count_body(t, acc):
            row = pos_scr[pl.ds(t, 1), :].reshape(128, 1)
            return acc + jnp.sum((row <= jlane).astype(jnp.float32), axis=0)

        counts = lax.fori_loop(
            0, NCH, count_body, jnp.zeros((128,), jnp.float32))
        base = bi * N
        perm_ref[0, 0, :] = jnp.minimum(counts.astype(jnp.int32), N - 1) + base

        n_fill = jnp.maximum(P - num_valid, 0)
        flat_iota = (lax.broadcasted_iota(jnp.int32, (NCH, 128), 0) * 128
                     + lax.broadcasted_iota(jnp.int32, (NCH, 128), 1))
        lane128 = lax.broadcasted_iota(jnp.int32, (128,), 0)

        def fill_body(t, _):
            msk = msk_scr[...]
            mv = jnp.min(msk)
            idx = jnp.min(jnp.where(msk == mv, flat_iota, N))
            msk_scr[...] = jnp.where(flat_iota == idx, 2.0, msk)
            slot = num_valid + t
            cur = perm_ref[0, 0, :]
            perm_ref[0, 0, :] = jnp.where(lane128 == slot, idx + base, cur)
            return 0

        lax.fori_loop(0, n_fill, fill_body, 0)


_SC_CORES = 2
_SC_SUBCORES = 16
_NW = _SC_CORES * _SC_SUBCORES
_GROWS = B * 128
_RPW = _GROWS // _NW


def _gather_kernel(table_hbm, idx_hbm, out_hbm, idx_v, rows_v, sem):
    wid = lax.axis_index("s") * _SC_CORES + lax.axis_index("c")
    rbase = wid * _RPW
    pltpu.sync_copy(idx_hbm.at[pl.ds(rbase, _RPW)], idx_v)
    pltpu.async_copy(table_hbm.at[idx_v], rows_v, sem).wait()
    pltpu.sync_copy(rows_v, out_hbm.at[pl.ds(rbase, _RPW)])


def kernel(encoder_tokens, W, b):
    wt = W.T
    b2 = b.reshape(1, NUM_CLASSES)

    mask3, perm = pl.pallas_call(
        _fused_kernel,
        grid=(B, NSTEP),
        in_specs=[
            pl.BlockSpec((1, ROWS_BLK, D), lambda bi, i: (bi, i, 0)),
            pl.BlockSpec((D, NUM_CLASSES), lambda bi, i: (0, 0)),
            pl.BlockSpec((1, NUM_CLASSES), lambda bi, i: (0, 0)),
        ],
        out_specs=[
            pl.BlockSpec((1, 1, ROWS_BLK), lambda bi, i: (bi, 0, i)),
            pl.BlockSpec((1, 1, 128), lambda bi, i: (bi, 0, 0)),
        ],
        out_shape=[
            jax.ShapeDtypeStruct((B, 1, N), jnp.float32),
            jax.ShapeDtypeStruct((B, 1, 128), jnp.int32),
        ],
        scratch_shapes=[
            pltpu.VMEM((NCH, 128), jnp.float32),
            pltpu.VMEM((NCH, 128), jnp.float32),
            pltpu.VMEM((NCH, 128), jnp.float32),
        ],
    )(encoder_tokens, wt, b2)
    selection_mask = mask3.reshape(B, N)
    idx_flat = perm.reshape(_GROWS)

    table = encoder_tokens.reshape(B * N, D)
    mesh = plsc.VectorSubcoreMesh(core_axis_name="c", subcore_axis_name="s")
    gathered = pl.kernel(
        _gather_kernel,
        out_type=jax.ShapeDtypeStruct((_GROWS, D), jnp.float32),
        mesh=mesh,
        scratch_types=[
            pltpu.VMEM((_RPW,), jnp.int32),
            pltpu.VMEM((_RPW, D), jnp.float32),
            pltpu.SemaphoreType.DMA,
        ],
    )(table, idx_flat)

    SADQ = gathered.reshape(B, 128, D)[:, :P, :]
    return (SADQ, selection_mask)


# speedup vs baseline: 1.1377x; 1.1377x over previous
import jax
import jax.numpy as jnp
from jax import lax
from jax.experimental import pallas as pl
from jax.experimental.pallas import tpu as pltpu
from jax.experimental.pallas import tpu_sc as plsc

B, N, D = 4, 8192, 512
P = 100
NUM_CLASSES = 91
ROWS_BLK = 1024
NSTEP = N // ROWS_BLK
NCH = N // 128
SUB = ROWS_BLK // 128
GROWS = 512


def _fused_kernel(x_ref, w_ref, b_ref, mask_ref, perm_ref,
                  s_scr, pos_scr, msk_scr, perm_scr):
    bi = pl.program_id(0)
    i = pl.program_id(1)
    x = x_ref[0]
    logits = lax.dot_general(
        x, w_ref[...], (((1,), (1,)), ((), ())),
        preferred_element_type=jnp.float32)
    logits = logits + b_ref[0]
    m = jnp.max(logits, axis=-1)
    s = jax.nn.sigmoid(m)
    soft = jax.nn.sigmoid((s - 0.5) / 0.1)
    hard = (s > 0.5).astype(jnp.float32)
    mask_ref[0, 0, :] = hard + soft - soft
    s_scr[pl.ds(i * SUB, SUB), :] = s.reshape(SUB, 128)

    @pl.when(i == NSTEP - 1)
    def _select():
        sall = s_scr[...]
        valid = sall > 0.5
        vf = valid.astype(jnp.float32)

        tri128 = (lax.broadcasted_iota(jnp.int32, (128, 128), 0)
                  <= lax.broadcasted_iota(jnp.int32, (128, 128), 1)
                  ).astype(jnp.float32)
        rowcum = jnp.dot(vf, tri128, preferred_element_type=jnp.float32)
        row_tot = rowcum[:, 127].reshape(1, NCH)
        tri64 = (lax.broadcasted_iota(jnp.int32, (NCH, NCH), 0)
                 <= lax.broadcasted_iota(jnp.int32, (NCH, NCH), 1)
                 ).astype(jnp.float32)
        inc = jnp.dot(row_tot, tri64, preferred_element_type=jnp.float32)
        excl = (inc - row_tot).reshape(NCH, 1)
        pos = rowcum + excl
        num_valid = inc[0, NCH - 1].astype(jnp.int32)

        pos_scr[...] = pos
        msk_scr[...] = jnp.where(valid, 2.0, sall)

        jlane = lax.broadcasted_iota(
            jnp.int32, (1, 128), 1).astype(jnp.float32)

        def count_body(t, acc):
            row = pos_scr[pl.ds(t, 1), :].reshape(128, 1)
            return acc + jnp.sum((row <= jlane).astype(jnp.float32), axis=0)

        counts = lax.fori_loop(
            0, NCH, count_body, jnp.zeros((128,), jnp.float32))
        base = bi * N
        perm_scr[pl.ds(bi, 1), :] = (
            jnp.minimum(counts.astype(jnp.int32), N - 1) + base
        ).reshape(1, 128)

        n_fill = jnp.maximum(P - num_valid, 0)
        flat_iota = (lax.broadcasted_iota(jnp.int32, (NCH, 128), 0) * 128
                     + lax.broadcasted_iota(jnp.int32, (NCH, 128), 1))
        lane128 = lax.broadcasted_iota(jnp.int32, (1, 128), 1)

        def fill_body(t, _):
            msk = msk_scr[...]
            mv = jnp.min(msk)
            idx = jnp.min(jnp.where(msk == mv, flat_iota, N))
            msk_scr[...] = jnp.where(flat_iota == idx, 2.0, msk)
            slot = num_valid + t
            cur = perm_scr[pl.ds(bi, 1), :]
            perm_scr[pl.ds(bi, 1), :] = jnp.where(
                lane128 == slot, idx + base, cur)
            return 0

        lax.fori_loop(0, n_fill, fill_body, 0)

        @pl.when(bi == B - 1)
        def _compose():
            parts = [perm_scr[k, :P] for k in range(B)]
            parts.append(jnp.zeros((GROWS - B * P,), jnp.int32))
            perm_ref[0, 0, :] = jnp.concatenate(parts)


_SC_CORES = 2
_SC_SUBCORES = 16
_NW = _SC_CORES * _SC_SUBCORES
_RPW = 16
_NW_ACT = (B * P) // _RPW


def _gather_kernel(table_hbm, idx_hbm, out_hbm, idx_v, rows_v, sem):
    wid = lax.axis_index("s") * _SC_CORES + lax.axis_index("c")

    @pl.when(wid < _NW_ACT)
    def _():
        rbase = wid * _RPW
        pltpu.sync_copy(idx_hbm.at[pl.ds(rbase, _RPW)], idx_v)
        pltpu.async_copy(table_hbm.at[idx_v], rows_v, sem).wait()
        pltpu.sync_copy(rows_v, out_hbm.at[pl.ds(rbase, _RPW)])


def kernel(encoder_tokens, W, b):
    b2 = b.reshape(1, NUM_CLASSES)

    mask3, perm = pl.pallas_call(
        _fused_kernel,
        grid=(B, NSTEP),
        in_specs=[
            pl.BlockSpec((1, ROWS_BLK, D), lambda bi, i: (bi, i, 0)),
            pl.BlockSpec((NUM_CLASSES, D), lambda bi, i: (0, 0)),
            pl.BlockSpec((1, NUM_CLASSES), lambda bi, i: (0, 0)),
        ],
        out_specs=[
            pl.BlockSpec((1, 1, ROWS_BLK), lambda bi, i: (bi, 0, i)),
            pl.BlockSpec((1, 1, GROWS), lambda bi, i: (0, 0, 0)),
        ],
        out_shape=[
            jax.ShapeDtypeStruct((B, 1, N), jnp.float32),
            jax.ShapeDtypeStruct((1, 1, GROWS), jnp.int32),
        ],
        scratch_shapes=[
            pltpu.VMEM((NCH, 128), jnp.float32),
            pltpu.VMEM((NCH, 128), jnp.float32),
            pltpu.VMEM((NCH, 128), jnp.float32),
            pltpu.VMEM((8, 128), jnp.int32),
        ],
    )(encoder_tokens, W, b2)
    selection_mask = mask3.reshape(B, N)
    idx_flat = perm.reshape(GROWS)

    table = encoder_tokens.reshape(B * N, D)
    mesh = plsc.VectorSubcoreMesh(core_axis_name="c", subcore_axis_name="s")
    gathered = pl.kernel(
        _gather_kernel,
        out_type=jax.ShapeDtypeStruct((B * P, D), jnp.float32),
        mesh=mesh,
        scratch_types=[
            pltpu.VMEM((_RPW,), jnp.int32),
            pltpu.VMEM((_RPW, D), jnp.float32),
            pltpu.SemaphoreType.DMA,
        ],
    )(table, idx_flat)

    SADQ = gathered.reshape(B, P, D)
    return (SADQ, selection_mask)


# speedup vs baseline: 1.3412x; 1.1788x over previous
import jax
import jax.numpy as jnp
from jax import lax
from jax.experimental import pallas as pl
from jax.experimental.pallas import tpu as pltpu
from jax.experimental.pallas import tpu_sc as plsc

B, N, D = 4, 8192, 512
P = 100
NUM_CLASSES = 91
ROWS_BLK = 1024
NSTEP = N // ROWS_BLK
NCH = N // 128
SUB = ROWS_BLK // 128
GROWS = 512


def _fused_kernel(x_ref, w_ref, b_ref, mask_ref, perm_ref,
                  s_scr, pos_scr, msk_scr, perm_scr):
    bi = pl.program_id(0)
    i = pl.program_id(1)
    x = x_ref[0]
    logitsT = lax.dot_general(
        w_ref[...], x, (((1,), (1,)), ((), ())),
        preferred_element_type=jnp.float32)
    logitsT = logitsT + b_ref[...]
    m = jnp.max(logitsT, axis=0)
    s = jax.nn.sigmoid(m)
    soft = jax.nn.sigmoid((s - 0.5) / 0.1)
    hard = (s > 0.5).astype(jnp.float32)
    mask_ref[0, 0, :] = hard + soft - soft
    s_scr[pl.ds(i * SUB, SUB), :] = s.reshape(SUB, 128)

    @pl.when(i == NSTEP - 1)
    def _select():
        sall = s_scr[...]
        valid = sall > 0.5
        vf = valid.astype(jnp.float32)

        tri128 = (lax.broadcasted_iota(jnp.int32, (128, 128), 0)
                  <= lax.broadcasted_iota(jnp.int32, (128, 128), 1)
                  ).astype(jnp.float32)
        rowcum = jnp.dot(vf, tri128, preferred_element_type=jnp.float32)
        row_tot = rowcum[:, 127].reshape(1, NCH)
        tri64 = (lax.broadcasted_iota(jnp.int32, (NCH, NCH), 0)
                 <= lax.broadcasted_iota(jnp.int32, (NCH, NCH), 1)
                 ).astype(jnp.float32)
        inc = jnp.dot(row_tot, tri64, preferred_element_type=jnp.float32)
        excl = (inc - row_tot).reshape(NCH, 1)
        pos = rowcum + excl
        num_valid = inc[0, NCH - 1].astype(jnp.int32)

        pos_scr[...] = pos
        msk_scr[...] = jnp.where(valid, 2.0, sall)

        jlane = lax.broadcasted_iota(
            jnp.int32, (1, 128), 1).astype(jnp.float32)

        def count_body(t, acc):
            row = pos_scr[pl.ds(t, 1), :].reshape(128, 1)
            return acc + jnp.sum((row <= jlane).astype(jnp.float32), axis=0)

        counts = lax.fori_loop(
            0, NCH, count_body, jnp.zeros((128,), jnp.float32))
        base = bi * N
        perm_scr[pl.ds(bi, 1), :] = (
            jnp.minimum(counts.astype(jnp.int32), N - 1) + base
        ).reshape(1, 128)

        n_fill = jnp.maximum(P - num_valid, 0)
        flat_iota = (lax.broadcasted_iota(jnp.int32, (NCH, 128), 0) * 128
                     + lax.broadcasted_iota(jnp.int32, (NCH, 128), 1))
        lane128 = lax.broadcasted_iota(jnp.int32, (1, 128), 1)

        def fill_body(t, _):
            msk = msk_scr[...]
            mv = jnp.min(msk)
            idx = jnp.min(jnp.where(msk == mv, flat_iota, N))
            msk_scr[...] = jnp.where(flat_iota == idx, 2.0, msk)
            slot = num_valid + t
            cur = perm_scr[pl.ds(bi, 1), :]
            perm_scr[pl.ds(bi, 1), :] = jnp.where(
                lane128 == slot, idx + base, cur)
            return 0

        lax.fori_loop(0, n_fill, fill_body, 0)

        @pl.when(bi == B - 1)
        def _compose():
            parts = [perm_scr[k, :P] for k in range(B)]
            parts.append(jnp.zeros((GROWS - B * P,), jnp.int32))
            perm_ref[0, 0, :] = jnp.concatenate(parts)


_SC_CORES = 2
_SC_SUBCORES = 16
_NW = _SC_CORES * _SC_SUBCORES
_RPW = 16
_NW_ACT = (B * P) // _RPW


def _gather_kernel(table_hbm, idx_hbm, out_hbm, idx_v, rows_v, sem):
    wid = lax.axis_index("s") * _SC_CORES + lax.axis_index("c")

    @pl.when(wid < _NW_ACT)
    def _():
        rbase = wid * _RPW
        pltpu.sync_copy(idx_hbm.at[pl.ds(rbase, _RPW)], idx_v)
        pltpu.async_copy(table_hbm.at[idx_v], rows_v, sem).wait()
        pltpu.sync_copy(rows_v, out_hbm.at[pl.ds(rbase, _RPW)])


def kernel(encoder_tokens, W, b):
    b2 = b.reshape(NUM_CLASSES, 1)

    mask3, perm = pl.pallas_call(
        _fused_kernel,
        grid=(B, NSTEP),
        in_specs=[
            pl.BlockSpec((1, ROWS_BLK, D), lambda bi, i: (bi, i, 0)),
            pl.BlockSpec((NUM_CLASSES, D), lambda bi, i: (0, 0)),
            pl.BlockSpec((NUM_CLASSES, 1), lambda bi, i: (0, 0)),
        ],
        out_specs=[
            pl.BlockSpec((1, 1, ROWS_BLK), lambda bi, i: (bi, 0, i)),
            pl.BlockSpec((1, 1, GROWS), lambda bi, i: (0, 0, 0)),
        ],
        out_shape=[
            jax.ShapeDtypeStruct((B, 1, N), jnp.float32),
            jax.ShapeDtypeStruct((1, 1, GROWS), jnp.int32),
        ],
        scratch_shapes=[
            pltpu.VMEM((NCH, 128), jnp.float32),
            pltpu.VMEM((NCH, 128), jnp.float32),
            pltpu.VMEM((NCH, 128), jnp.float32),
            pltpu.VMEM((8, 128), jnp.int32),
        ],
    )(encoder_tokens, W, b2)
    selection_mask = mask3.reshape(B, N)
    idx_flat = perm.reshape(GROWS)

    table = encoder_tokens.reshape(B * N, D)
    mesh = plsc.VectorSubcoreMesh(core_axis_name="c", subcore_axis_name="s")
    gathered = pl.kernel(
        _gather_kernel,
        out_type=jax.ShapeDtypeStruct((B * P, D), jnp.float32),
        mesh=mesh,
        scratch_types=[
            pltpu.VMEM((_RPW,), jnp.int32),
            pltpu.VMEM((_RPW, D), jnp.float32),
            pltpu.SemaphoreType.DMA,
        ],
    )(table, idx_flat)

    SADQ = gathered.reshape(B, P, D)
    return (SADQ, selection_mask)


# speedup vs baseline: 1.5149x; 1.1295x over previous
import jax
import jax.numpy as jnp
from jax import lax
from jax.experimental import pallas as pl
from jax.experimental.pallas import tpu as pltpu
from jax.experimental.pallas import tpu_sc as plsc

B, N, D = 4, 8192, 512
P = 100
NUM_CLASSES = 91
ROWS_BLK = 2048
NSTEP = N // ROWS_BLK
NCH = N // 128
SUB = ROWS_BLK // 128
GROWS = 512


def _fused_kernel(x_ref, w_ref, b_ref, mask_ref, perm_ref,
                  s_scr, pos_scr, msk_scr, perm_scr):
    bi = pl.program_id(0)
    i = pl.program_id(1)
    x = x_ref[0]
    logitsT = lax.dot_general(
        w_ref[...], x, (((1,), (1,)), ((), ())),
        preferred_element_type=jnp.float32)
    logitsT = logitsT + b_ref[...]
    m = jnp.max(logitsT, axis=0)
    s = jax.nn.sigmoid(m)
    soft = jax.nn.sigmoid((s - 0.5) / 0.1)
    hard = (s > 0.5).astype(jnp.float32)
    mask_ref[0, 0, :] = hard + soft - soft
    s_scr[pl.ds(i * SUB, SUB), :] = s.reshape(SUB, 128)

    @pl.when(i == NSTEP - 1)
    def _select():
        sall = s_scr[...]
        valid = sall > 0.5
        vf = valid.astype(jnp.float32)

        tri128 = (lax.broadcasted_iota(jnp.int32, (128, 128), 0)
                  <= lax.broadcasted_iota(jnp.int32, (128, 128), 1)
                  ).astype(jnp.float32)
        rowcum = jnp.dot(vf, tri128, preferred_element_type=jnp.float32)
        row_tot = rowcum[:, 127].reshape(1, NCH)
        tri64 = (lax.broadcasted_iota(jnp.int32, (NCH, NCH), 0)
                 <= lax.broadcasted_iota(jnp.int32, (NCH, NCH), 1)
                 ).astype(jnp.float32)
        inc = jnp.dot(row_tot, tri64, preferred_element_type=jnp.float32)
        excl = (inc - row_tot).reshape(NCH, 1)
        pos = rowcum + excl
        num_valid = inc[0, NCH - 1].astype(jnp.int32)

        pos_scr[...] = pos
        msk_scr[...] = jnp.where(valid, 2.0, sall)

        jlane = lax.broadcasted_iota(
            jnp.int32, (1, 128), 1).astype(jnp.float32)

        def count_body(t, acc):
            row = pos_scr[pl.ds(t, 1), :].reshape(128, 1)
            return acc + jnp.sum((row <= jlane).astype(jnp.float32), axis=0)

        counts = lax.fori_loop(
            0, NCH, count_body, jnp.zeros((128,), jnp.float32))
        base = bi * N
        perm_scr[pl.ds(bi, 1), :] = (
            jnp.minimum(counts.astype(jnp.int32), N - 1) + base
        ).reshape(1, 128)

        n_fill = jnp.maximum(P - num_valid, 0)
        flat_iota = (lax.broadcasted_iota(jnp.int32, (NCH, 128), 0) * 128
                     + lax.broadcasted_iota(jnp.int32, (NCH, 128), 1))
        lane128 = lax.broadcasted_iota(jnp.int32, (1, 128), 1)

        def fill_body(t, _):
            msk = msk_scr[...]
            mv = jnp.min(msk)
            idx = jnp.min(jnp.where(msk == mv, flat_iota, N))
            msk_scr[...] = jnp.where(flat_iota == idx, 2.0, msk)
            slot = num_valid + t
            cur = perm_scr[pl.ds(bi, 1), :]
            perm_scr[pl.ds(bi, 1), :] = jnp.where(
                lane128 == slot, idx + base, cur)
            return 0

        lax.fori_loop(0, n_fill, fill_body, 0)

        @pl.when(bi == B - 1)
        def _compose():
            parts = [perm_scr[k, :P] for k in range(B)]
            parts.append(jnp.zeros((GROWS - B * P,), jnp.int32))
            perm_ref[0, 0, :] = jnp.concatenate(parts)


_SC_CORES = 2
_SC_SUBCORES = 16
_NW = _SC_CORES * _SC_SUBCORES
_RPW = 16
_NW_ACT = (B * P) // _RPW


def _gather_kernel(table_hbm, idx_hbm, out_hbm, idx_v, rows_v, sem):
    wid = lax.axis_index("s") * _SC_CORES + lax.axis_index("c")

    @pl.when(wid < _NW_ACT)
    def _():
        rbase = wid * _RPW
        pltpu.sync_copy(idx_hbm.at[pl.ds(rbase, _RPW)], idx_v)
        pltpu.async_copy(table_hbm.at[idx_v], rows_v, sem).wait()
        pltpu.sync_copy(rows_v, out_hbm.at[pl.ds(rbase, _RPW)])


def kernel(encoder_tokens, W, b):
    b2 = b.reshape(NUM_CLASSES, 1)

    mask3, perm = pl.pallas_call(
        _fused_kernel,
        grid=(B, NSTEP),
        in_specs=[
            pl.BlockSpec((1, ROWS_BLK, D), lambda bi, i: (bi, i, 0)),
            pl.BlockSpec((NUM_CLASSES, D), lambda bi, i: (0, 0)),
            pl.BlockSpec((NUM_CLASSES, 1), lambda bi, i: (0, 0)),
        ],
        out_specs=[
            pl.BlockSpec((1, 1, ROWS_BLK), lambda bi, i: (bi, 0, i)),
            pl.BlockSpec((1, 1, GROWS), lambda bi, i: (0, 0, 0)),
        ],
        out_shape=[
            jax.ShapeDtypeStruct((B, 1, N), jnp.float32),
            jax.ShapeDtypeStruct((1, 1, GROWS), jnp.int32),
        ],
        scratch_shapes=[
            pltpu.VMEM((NCH, 128), jnp.float32),
            pltpu.VMEM((NCH, 128), jnp.float32),
            pltpu.VMEM((NCH, 128), jnp.float32),
            pltpu.VMEM((8, 128), jnp.int32),
        ],
    )(encoder_tokens, W, b2)
    selection_mask = mask3.reshape(B, N)
    idx_flat = perm.reshape(GROWS)

    table = encoder_tokens.reshape(B * N, D)
    mesh = plsc.VectorSubcoreMesh(core_axis_name="c", subcore_axis_name="s")
    gathered = pl.kernel(
        _gather_kernel,
        out_type=jax.ShapeDtypeStruct((B * P, D), jnp.float32),
        mesh=mesh,
        scratch_types=[
            pltpu.VMEM((_RPW,), jnp.int32),
            pltpu.VMEM((_RPW, D), jnp.float32),
            pltpu.SemaphoreType.DMA,
        ],
    )(table, idx_flat)

    SADQ = gathered.reshape(B, P, D)
    return (SADQ, selection_mask)


# speedup vs baseline: 1.5926x; 1.0513x over previous
import jax
import jax.numpy as jnp
from jax import lax
from jax.experimental import pallas as pl
from jax.experimental.pallas import tpu as pltpu
from jax.experimental.pallas import tpu_sc as plsc

B, N, D = 4, 8192, 512
P = 100
NUM_CLASSES = 91
ROWS_BLK = 4096
NSTEP = N // ROWS_BLK
NCH = N // 128
SUB = ROWS_BLK // 128
GROWS = 512


def _fused_kernel(x_ref, w_ref, b_ref, mask_ref, perm_ref,
                  s_scr, pos_scr, msk_scr, perm_scr):
    bi = pl.program_id(0)
    i = pl.program_id(1)
    x = x_ref[0]
    logitsT = lax.dot_general(
        w_ref[...], x, (((1,), (1,)), ((), ())),
        preferred_element_type=jnp.float32)
    logitsT = logitsT + b_ref[...]
    m = jnp.max(logitsT, axis=0)
    s = jax.nn.sigmoid(m)
    soft = jax.nn.sigmoid((s - 0.5) / 0.1)
    hard = (s > 0.5).astype(jnp.float32)
    mask_ref[0, 0, :] = hard + soft - soft
    s_scr[pl.ds(i * SUB, SUB), :] = s.reshape(SUB, 128)

    @pl.when(i == NSTEP - 1)
    def _select():
        sall = s_scr[...]
        valid = sall > 0.5
        vf = valid.astype(jnp.float32)

        tri128 = (lax.broadcasted_iota(jnp.int32, (128, 128), 0)
                  <= lax.broadcasted_iota(jnp.int32, (128, 128), 1)
                  ).astype(jnp.float32)
        rowcum = jnp.dot(vf, tri128, preferred_element_type=jnp.float32)
        row_tot = rowcum[:, 127].reshape(1, NCH)
        tri64 = (lax.broadcasted_iota(jnp.int32, (NCH, NCH), 0)
                 <= lax.broadcasted_iota(jnp.int32, (NCH, NCH), 1)
                 ).astype(jnp.float32)
        inc = jnp.dot(row_tot, tri64, preferred_element_type=jnp.float32)
        excl = (inc - row_tot).reshape(NCH, 1)
        pos = rowcum + excl
        num_valid = inc[0, NCH - 1].astype(jnp.int32)

        pos_scr[...] = pos
        msk_scr[...] = jnp.where(valid, 2.0, sall)

        jlane = lax.broadcasted_iota(
            jnp.int32, (1, 128), 1).astype(jnp.float32)

        def count_body(t, acc):
            row = pos_scr[pl.ds(t, 1), :].reshape(128, 1)
            return acc + jnp.sum((row <= jlane).astype(jnp.float32), axis=0)

        counts = lax.fori_loop(
            0, NCH, count_body, jnp.zeros((128,), jnp.float32))
        base = bi * N
        perm_scr[pl.ds(bi, 1), :] = (
            jnp.minimum(counts.astype(jnp.int32), N - 1) + base
        ).reshape(1, 128)

        n_fill = jnp.maximum(P - num_valid, 0)
        flat_iota = (lax.broadcasted_iota(jnp.int32, (NCH, 128), 0) * 128
                     + lax.broadcasted_iota(jnp.int32, (NCH, 128), 1))
        lane128 = lax.broadcasted_iota(jnp.int32, (1, 128), 1)

        def fill_body(t, _):
            msk = msk_scr[...]
            mv = jnp.min(msk)
            idx = jnp.min(jnp.where(msk == mv, flat_iota, N))
            msk_scr[...] = jnp.where(flat_iota == idx, 2.0, msk)
            slot = num_valid + t
            cur = perm_scr[pl.ds(bi, 1), :]
            perm_scr[pl.ds(bi, 1), :] = jnp.where(
                lane128 == slot, idx + base, cur)
            return 0

        lax.fori_loop(0, n_fill, fill_body, 0)

        @pl.when(bi == B - 1)
        def _compose():
            parts = [perm_scr[k, :P] for k in range(B)]
            parts.append(jnp.zeros((GROWS - B * P,), jnp.int32))
            perm_ref[0, 0, :] = jnp.concatenate(parts)


_SC_CORES = 2
_SC_SUBCORES = 16
_NW = _SC_CORES * _SC_SUBCORES
_RPW = 16
_NW_ACT = (B * P) // _RPW


def _gather_kernel(table_hbm, idx_hbm, out_hbm, idx_v, rows_v, sem):
    wid = lax.axis_index("s") * _SC_CORES + lax.axis_index("c")

    @pl.when(wid < _NW_ACT)
    def _():
        rbase = wid * _RPW
        pltpu.sync_copy(idx_hbm.at[pl.ds(rbase, _RPW)], idx_v)
        pltpu.async_copy(table_hbm.at[idx_v], rows_v, sem).wait()
        pltpu.sync_copy(rows_v, out_hbm.at[pl.ds(rbase, _RPW)])


def kernel(encoder_tokens, W, b):
    b2 = b.reshape(NUM_CLASSES, 1)

    mask3, perm = pl.pallas_call(
        _fused_kernel,
        grid=(B, NSTEP),
        in_specs=[
            pl.BlockSpec((1, ROWS_BLK, D), lambda bi, i: (bi, i, 0)),
            pl.BlockSpec((NUM_CLASSES, D), lambda bi, i: (0, 0)),
            pl.BlockSpec((NUM_CLASSES, 1), lambda bi, i: (0, 0)),
        ],
        out_specs=[
            pl.BlockSpec((1, 1, ROWS_BLK), lambda bi, i: (bi, 0, i)),
            pl.BlockSpec((1, 1, GROWS), lambda bi, i: (0, 0, 0)),
        ],
        out_shape=[
            jax.ShapeDtypeStruct((B, 1, N), jnp.float32),
            jax.ShapeDtypeStruct((1, 1, GROWS), jnp.int32),
        ],
        scratch_shapes=[
            pltpu.VMEM((NCH, 128), jnp.float32),
            pltpu.VMEM((NCH, 128), jnp.float32),
            pltpu.VMEM((NCH, 128), jnp.float32),
            pltpu.VMEM((8, 128), jnp.int32),
        ],
    )(encoder_tokens, W, b2)
    selection_mask = mask3.reshape(B, N)
    idx_flat = perm.reshape(GROWS)

    table = encoder_tokens.reshape(B * N, D)
    mesh = plsc.VectorSubcoreMesh(core_axis_name="c", subcore_axis_name="s")
    gathered = pl.kernel(
        _gather_kernel,
        out_type=jax.ShapeDtypeStruct((B * P, D), jnp.float32),
        mesh=mesh,
        scratch_types=[
            pltpu.VMEM((_RPW,), jnp.int32),
            pltpu.VMEM((_RPW, D), jnp.float32),
            pltpu.SemaphoreType.DMA,
        ],
    )(table, idx_flat)

    SADQ = gathered.reshape(B, P, D)
    return (SADQ, selection_mask)


# speedup vs baseline: 1.7166x; 1.0778x over previous
import jax
import jax.numpy as jnp
from jax import lax
from jax.experimental import pallas as pl
from jax.experimental.pallas import tpu as pltpu
from jax.experimental.pallas import tpu_sc as plsc

B, N, D = 4, 8192, 512
P = 100
NUM_CLASSES = 91
ROWS_BLK = 8192
NSTEP = N // ROWS_BLK
NCH = N // 128
SUB = ROWS_BLK // 128
GROWS = 512


def _fused_kernel(x_ref, w_ref, b_ref, mask_ref, perm_ref,
                  s_scr, pos_scr, msk_scr, perm_scr):
    bi = pl.program_id(0)
    i = pl.program_id(1)
    x = x_ref[0]
    logitsT = lax.dot_general(
        w_ref[...], x, (((1,), (1,)), ((), ())),
        preferred_element_type=jnp.float32)
    logitsT = logitsT + b_ref[...]
    m = jnp.max(logitsT, axis=0)
    s = jax.nn.sigmoid(m)
    soft = jax.nn.sigmoid((s - 0.5) / 0.1)
    hard = (s > 0.5).astype(jnp.float32)
    mask_ref[0, 0, :] = hard + soft - soft
    s_scr[pl.ds(i * SUB, SUB), :] = s.reshape(SUB, 128)

    @pl.when(i == NSTEP - 1)
    def _select():
        sall = s_scr[...]
        valid = sall > 0.5
        vf = valid.astype(jnp.float32)

        tri128 = (lax.broadcasted_iota(jnp.int32, (128, 128), 0)
                  <= lax.broadcasted_iota(jnp.int32, (128, 128), 1)
                  ).astype(jnp.float32)
        rowcum = jnp.dot(vf, tri128, preferred_element_type=jnp.float32)
        row_tot = rowcum[:, 127].reshape(1, NCH)
        tri64 = (lax.broadcasted_iota(jnp.int32, (NCH, NCH), 0)
                 <= lax.broadcasted_iota(jnp.int32, (NCH, NCH), 1)
                 ).astype(jnp.float32)
        inc = jnp.dot(row_tot, tri64, preferred_element_type=jnp.float32)
        excl = (inc - row_tot).reshape(NCH, 1)
        pos = rowcum + excl
        num_valid = inc[0, NCH - 1].astype(jnp.int32)

        pos_scr[...] = pos
        msk_scr[...] = jnp.where(valid, 2.0, sall)

        jlane = lax.broadcasted_iota(
            jnp.int32, (1, 128), 1).astype(jnp.float32)

        def count_body(t, acc):
            row = pos_scr[pl.ds(t, 1), :].reshape(128, 1)
            return acc + jnp.sum((row <= jlane).astype(jnp.float32), axis=0)

        counts = lax.fori_loop(
            0, NCH, count_body, jnp.zeros((128,), jnp.float32))
        base = bi * N
        perm_scr[pl.ds(bi, 1), :] = (
            jnp.minimum(counts.astype(jnp.int32), N - 1) + base
        ).reshape(1, 128)

        n_fill = jnp.maximum(P - num_valid, 0)
        flat_iota = (lax.broadcasted_iota(jnp.int32, (NCH, 128), 0) * 128
                     + lax.broadcasted_iota(jnp.int32, (NCH, 128), 1))
        lane128 = lax.broadcasted_iota(jnp.int32, (1, 128), 1)

        def fill_body(t, _):
            msk = msk_scr[...]
            mv = jnp.min(msk)
            idx = jnp.min(jnp.where(msk == mv, flat_iota, N))
            msk_scr[...] = jnp.where(flat_iota == idx, 2.0, msk)
            slot = num_valid + t
            cur = perm_scr[pl.ds(bi, 1), :]
            perm_scr[pl.ds(bi, 1), :] = jnp.where(
                lane128 == slot, idx + base, cur)
            return 0

        lax.fori_loop(0, n_fill, fill_body, 0)

        @pl.when(bi == B - 1)
        def _compose():
            parts = [perm_scr[k, :P] for k in range(B)]
            parts.append(jnp.zeros((GROWS - B * P,), jnp.int32))
            perm_ref[0, 0, :] = jnp.concatenate(parts)


_SC_CORES = 2
_SC_SUBCORES = 16
_NW = _SC_CORES * _SC_SUBCORES
_RPW = 16
_NW_ACT = (B * P) // _RPW


def _gather_kernel(table_hbm, idx_hbm, out_hbm, idx_v, rows_v, sem):
    wid = lax.axis_index("s") * _SC_CORES + lax.axis_index("c")

    @pl.when(wid < _NW_ACT)
    def _():
        rbase = wid * _RPW
        pltpu.sync_copy(idx_hbm.at[pl.ds(rbase, _RPW)], idx_v)
        pltpu.async_copy(table_hbm.at[idx_v], rows_v, sem).wait()
        pltpu.sync_copy(rows_v, out_hbm.at[pl.ds(rbase, _RPW)])


def kernel(encoder_tokens, W, b):
    b2 = b.reshape(NUM_CLASSES, 1)

    mask3, perm = pl.pallas_call(
        _fused_kernel,
        grid=(B, NSTEP),
        in_specs=[
            pl.BlockSpec((1, ROWS_BLK, D), lambda bi, i: (bi, i, 0)),
            pl.BlockSpec((NUM_CLASSES, D), lambda bi, i: (0, 0)),
            pl.BlockSpec((NUM_CLASSES, 1), lambda bi, i: (0, 0)),
        ],
        out_specs=[
            pl.BlockSpec((1, 1, ROWS_BLK), lambda bi, i: (bi, 0, i)),
            pl.BlockSpec((1, 1, GROWS), lambda bi, i: (0, 0, 0)),
        ],
        out_shape=[
            jax.ShapeDtypeStruct((B, 1, N), jnp.float32),
            jax.ShapeDtypeStruct((1, 1, GROWS), jnp.int32),
        ],
        scratch_shapes=[
            pltpu.VMEM((NCH, 128), jnp.float32),
            pltpu.VMEM((NCH, 128), jnp.float32),
            pltpu.VMEM((NCH, 128), jnp.float32),
            pltpu.VMEM((8, 128), jnp.int32),
        ],
    )(encoder_tokens, W, b2)
    selection_mask = mask3.reshape(B, N)
    idx_flat = perm.reshape(GROWS)

    table = encoder_tokens.reshape(B * N, D)
    mesh = plsc.VectorSubcoreMesh(core_axis_name="c", subcore_axis_name="s")
    gathered = pl.kernel(
        _gather_kernel,
        out_type=jax.ShapeDtypeStruct((B * P, D), jnp.float32),
        mesh=mesh,
        scratch_types=[
            pltpu.VMEM((_RPW,), jnp.int32),
            pltpu.VMEM((_RPW, D), jnp.float32),
            pltpu.SemaphoreType.DMA,
        ],
    )(table, idx_flat)

    SADQ = gathered.reshape(B, P, D)
    return (SADQ, selection_mask)
